# Initial kernel scaffold; baseline (speedup 1.0000x reference)
import jax
import jax.numpy as jnp
from jax.experimental import pallas as pl


def kernel(adj_indices, adj_values, image_adj_indices, image_adj_values, text_adj_indices, text_adj_values, u_embs, i_embs, image_embedding, text_embedding, image_W, image_b, text_W, text_b, modal_weight):
    raise NotImplementedError("write your pallas kernel here")


# speedup vs baseline: 2.6008x; 2.6008x over previous
import functools

import jax
import jax.numpy as jnp
from jax import lax
from jax.experimental import pallas as pl
from jax.experimental.pallas import tpu as pltpu
from jax.experimental.pallas import tpu_sc as plsc

_USER_NUM = 30000
_ITEM_NUM = 20000
_N = _USER_NUM + _ITEM_NUM
_D = 64
_HD = 32
_E = 800000
_MODAL_ADJ_W = 0.2
_RESID_W = 0.5

_NC = 2
_NS = 16
_L = 16

_RUN_PASSES = 3
_DO_SCATTER = True
_PIPELINED = False
_DO_CRUNCH = True

_C = 128
_EPT_CH = -(-_E // (_NS * _C))
_EPT = _EPT_CH * _C
_E_PAD = _EPT * _NS
_N_CH = -(-_N // (_NS * _C))
_STRIPE = _N_CH * _C
_N_PAD = _STRIPE * _NS


def _sc_kernel_body(
    rows_a, cols_a, vals_a,
    rows_i, cols_i, vals_i,
    rows_t, cols_t, vals_t,
    tb_base, tb_img, tb_txt,
    scales_hbm,
    out_hbm,
    modal_hbm,
    accum,
    gidx0, gidx1,
    ridx0, ridx1,
    vbuf0, vbuf1,
    rbuf0, rbuf1,
    wbuf,
    scales_v,
    sem0, sem1,
):
  gidxs, ridxs = (gidx0, gidx1), (ridx0, ridx1)
  vbufs, rbufs, sems = (vbuf0, vbuf1), (rbuf0, rbuf1), (sem0, sem1)
  h = lax.axis_index("c")
  s = lax.axis_index("s")
  ebase = s * _EPT
  stripe0 = s * _STRIPE

  pltpu.sync_copy(scales_hbm, scales_v)

  zv = jnp.zeros((_L,), jnp.float32)

  @pl.loop(0, _C)
  def _zero_wbuf(r):
    wbuf[r, pl.ds(0, _L)] = zv
    wbuf[r, pl.ds(_L, _L)] = zv

  @pl.loop(0, _N_CH)
  def _zero_accum(k):
    pltpu.sync_copy(wbuf, accum.at[pl.ds(stripe0 + k * _C, _C)])

  plsc.subcore_barrier()

  def emit_pass(rows_hbm, cols_hbm, vals_hbm, table_hbm, scale_row, half_rows):
    scale_v = scales_v[scale_row]
    hoff = jnp.full((_L,), h * half_rows, jnp.int32)

    def issue(g, b):
      off = ebase + g * _C
      gidx, ridx, vbuf = gidxs[b], ridxs[b], vbufs[b]
      pltpu.sync_copy(cols_hbm.at[pl.ds(off, _C)], gidx)
      for j in range(_C // _L):
        gidx[pl.ds(j * _L, _L)] = gidx[pl.ds(j * _L, _L)] + hoff
      pltpu.sync_copy(rows_hbm.at[pl.ds(off, _C)], ridx)
      pltpu.sync_copy(vals_hbm.at[pl.ds(off, _C)], vbuf)
      pltpu.async_copy(table_hbm.at[gidx], rbufs[b], sems[b])

    def crunch(b, wait=True):
      gidx, ridx, vbuf, rbuf = gidxs[b], ridxs[b], vbufs[b], rbufs[b]
      if wait:
        pltpu.make_async_copy(table_hbm.at[gidx], rbuf, sems[b]).wait()

      @pl.loop(0, _C // _L)
      def _grp(grp):
        vv = vbuf[pl.ds(grp * _L, _L)] * scale_v
        vbuf[pl.ds(grp * _L, _L)] = vv
        for i in range(_L):
          e = grp * _L + i
          vb = plsc.load_gather(vbuf, [jnp.full((_L,), e, jnp.int32)])
          rbuf[e, pl.ds(0, _L)] = rbuf[e, pl.ds(0, _L)] * vb
          rbuf[e, pl.ds(_L, _L)] = rbuf[e, pl.ds(_L, _L)] * vb

      if _DO_SCATTER:
        pltpu.sync_copy(rbuf, accum.at[ridx], add=True)

    if _PIPELINED:
      issue(0, 0)
      issue(1, 1)

      @pl.loop(0, _EPT_CH // 2)
      def _chunks(p):
        for b in range(2):
          g = p * 2 + b
          crunch(b)

          @pl.when(g + 2 < _EPT_CH)
          def _():
            issue(g + 2, b)
    else:
      @pl.loop(0, _EPT_CH)
      def _chunks(g):
        issue(g, 0)
        pltpu.make_async_copy(table_hbm.at[gidxs[0]], rbufs[0], sems[0]).wait()
        if _DO_CRUNCH:
          crunch(0, wait=False)

  if _RUN_PASSES >= 1:
    emit_pass(rows_a, cols_a, vals_a, tb_base, 0, _N)
  if _RUN_PASSES >= 2:
    emit_pass(rows_i, cols_i, vals_i, tb_img, 1, _N)
    emit_pass(rows_t, cols_t, vals_t, tb_txt, 2, _N)
  plsc.subcore_barrier()

  hN = h * _N_PAD

  @pl.loop(0, _N_CH)
  def _modal_out(k):
    base = stripe0 + k * _C
    pltpu.sync_copy(accum.at[pl.ds(base, _C)], wbuf)
    pltpu.sync_copy(wbuf, modal_hbm.at[pl.ds(hN + base, _C)])

    @pl.loop(0, _C)
    def _scale(r):
      wbuf[r, pl.ds(0, _L)] = wbuf[r, pl.ds(0, _L)] * (1.0 + _RESID_W)
      wbuf[r, pl.ds(_L, _L)] = wbuf[r, pl.ds(_L, _L)] * (1.0 + _RESID_W)

    pltpu.sync_copy(wbuf, accum.at[pl.ds(base, _C)])

  plsc.subcore_barrier()

  if _RUN_PASSES >= 3:
    emit_pass(rows_a, cols_a, vals_a, modal_hbm, 3, _N_PAD)
  plsc.subcore_barrier()

  @pl.loop(0, _N_CH)
  def _final_out(k):
    base = stripe0 + k * _C
    pltpu.sync_copy(accum.at[pl.ds(base, _C)], wbuf)
    pltpu.sync_copy(wbuf, out_hbm.at[h, pl.ds(base, _C)])


def _sc_spmm(rows_a, cols_a, vals_a, rows_i, cols_i, vals_i,
             rows_t, cols_t, vals_t, tb_base, tb_img, tb_txt, scales):
  mesh = plsc.VectorSubcoreMesh(
      core_axis_name="c", subcore_axis_name="s",
      num_cores=_NC, num_subcores=_NS)
  fn = pl.kernel(
      _sc_kernel_body,
      out_type=[
          jax.ShapeDtypeStruct((_NC, _N_PAD, _HD), jnp.float32),
          jax.ShapeDtypeStruct((_NC * _N_PAD, _HD), jnp.float32),
      ],
      mesh=mesh,
      compiler_params=pltpu.CompilerParams(
          needs_layout_passes=False, use_tc_tiling_on_sc=False),
      scratch_types=[
          pltpu.VMEM_SHARED((_N_PAD, _HD), jnp.float32),
          pltpu.VMEM((_C,), jnp.int32),
          pltpu.VMEM((_C,), jnp.int32),
          pltpu.VMEM((_C,), jnp.int32),
          pltpu.VMEM((_C,), jnp.int32),
          pltpu.VMEM((_C,), jnp.float32),
          pltpu.VMEM((_C,), jnp.float32),
          pltpu.VMEM((_C, _HD), jnp.float32),
          pltpu.VMEM((_C, _HD), jnp.float32),
          pltpu.VMEM((_C, _HD), jnp.float32),
          pltpu.VMEM((4, _L), jnp.float32),
          pltpu.SemaphoreType.DMA,
          pltpu.SemaphoreType.DMA,
      ],
  )
  out, _ = fn(rows_a, cols_a, vals_a, rows_i, cols_i, vals_i,
              rows_t, cols_t, vals_t, tb_base, tb_img, tb_txt, scales)
  return out


def _proj_body(x_ref, w_ref, b_ref, o_ref):
  y = jnp.dot(x_ref[...], w_ref[...], preferred_element_type=jnp.float32)
  y = y + b_ref[...]
  norm = jnp.sqrt(jnp.sum(y * y, axis=1, keepdims=True))
  o_ref[...] = y / jnp.maximum(norm, 1e-12)


def _proj_l2(x, w, b):
  m, k = x.shape
  bm = 400
  return pl.pallas_call(
      _proj_body,
      grid=(m // bm,),
      in_specs=[
          pl.BlockSpec((bm, k), lambda i: (i, 0)),
          pl.BlockSpec((k, _D), lambda i: (0, 0)),
          pl.BlockSpec((1, _D), lambda i: (0, 0)),
      ],
      out_specs=pl.BlockSpec((bm, _D), lambda i: (i, 0)),
      out_shape=jax.ShapeDtypeStruct((m, _D), jnp.float32),
  )(x, w, b.reshape(1, _D))


def _pad_edges(indices, values):
  rows = jnp.pad(indices[0], (0, _E_PAD - _E))
  cols = jnp.pad(indices[1], (0, _E_PAD - _E))
  vals = jnp.pad(values, (0, _E_PAD - _E))
  return rows, cols, vals


def _halves_table(top, bot):
  return jnp.concatenate(
      [top[:, :_HD], bot[:, :_HD], top[:, _HD:], bot[:, _HD:]], axis=0)


def kernel(adj_indices, adj_values, image_adj_indices, image_adj_values,
           text_adj_indices, text_adj_values, u_embs, i_embs,
           image_embedding, text_embedding, image_W, image_b,
           text_W, text_b, modal_weight):
  img_feats = _proj_l2(image_embedding, image_W, image_b)
  txt_feats = _proj_l2(text_embedding, text_W, text_b)

  weight = jax.nn.softmax(modal_weight, axis=-1)
  ones = jnp.ones((_L,), jnp.float32)
  scales = jnp.stack([
      ones,
      _MODAL_ADJ_W * weight[0] * ones,
      _MODAL_ADJ_W * weight[1] * ones,
      ones,
  ])

  tb_base = _halves_table(u_embs, i_embs)
  tb_img = _halves_table(u_embs, img_feats)
  tb_txt = _halves_table(u_embs, txt_feats)

  rows_a, cols_a, vals_a = _pad_edges(adj_indices, adj_values)
  rows_i, cols_i, vals_i = _pad_edges(image_adj_indices, image_adj_values)
  rows_t, cols_t, vals_t = _pad_edges(text_adj_indices, text_adj_values)

  out = _sc_spmm(rows_a, cols_a, vals_a, rows_i, cols_i, vals_i,
                 rows_t, cols_t, vals_t, tb_base, tb_img, tb_txt, scales)

  final = jnp.concatenate([out[0, :_N, :], out[1, :_N, :]], axis=1)
  return final[:_USER_NUM], final[_USER_NUM:]


# speedup vs baseline: 3.3676x; 1.2948x over previous
import jax
import jax.numpy as jnp
from jax import lax
from jax.experimental import pallas as pl
from jax.experimental.pallas import tpu as pltpu
from jax.experimental.pallas import tpu_sc as plsc

_USER_NUM = 30000
_ITEM_NUM = 20000
_N = _USER_NUM + _ITEM_NUM
_D = 64
_HD = 32
_E = 800000
_MODAL_ADJ_W = 0.2
_RESID_W = 0.5

_NC = 2
_NS = 16
_L = 16

_C = 128
_EPT_CH = -(-_E // (_NS * _C))
_EPT = _EPT_CH * _C
_E_PAD = _EPT * _NS
_N_CH = -(-_N // (_NS * _C))
_STRIPE = _N_CH * _C
_N_PAD = _STRIPE * _NS


def _sc_kernel_body(
    pk_a, pk_i, pk_t,
    tb_base, tb_img, tb_txt,
    scales_hbm,
    out_hbm,
    modal_hbm,
    accum,
    ebuf,
    ridx,
    vbuf,
    rbuf,
    wbuf,
    scales_v,
    gsem,
):
  h = lax.axis_index("c")
  s = lax.axis_index("s")
  stripe0 = s * _STRIPE

  pltpu.sync_copy(scales_hbm, scales_v)

  zv = jnp.zeros((_L,), jnp.float32)

  @pl.loop(0, _C)
  def _zero_wbuf(r):
    wbuf[r, pl.ds(0, _L)] = zv
    wbuf[r, pl.ds(_L, _L)] = zv

  @pl.loop(0, _N_CH)
  def _zero_accum(k):
    pltpu.sync_copy(wbuf, accum.at[pl.ds(stripe0 + k * _C, _C)])

  plsc.subcore_barrier()

  def emit_pass(packed_hbm, table_hbm, scale_row, half_rows):
    scale_v = scales_v[scale_row]
    hoff = jnp.full((_L,), h * half_rows, jnp.int32)
    ch0 = s * _EPT_CH

    @pl.loop(0, _EPT_CH)
    def _chunks(g):
      off = (ch0 + g) * (3 * _C)
      pltpu.sync_copy(packed_hbm.at[pl.ds(off, 3 * _C)], ebuf)
      for j in range(_C // _L):
        ebuf[pl.ds(j * _L, _L)] = ebuf[pl.ds(j * _L, _L)] + hoff
      gidx = ebuf.at[pl.ds(0, _C)]
      pltpu.async_copy(table_hbm.at[gidx], rbuf, gsem)
      pltpu.make_async_copy(table_hbm.at[gidx], rbuf, gsem).wait()

      @pl.loop(0, _C // _L)
      def _grp(grp):
        o = grp * _L
        vv = plsc.bitcast(ebuf[pl.ds(2 * _C + o, _L)], jnp.float32)
        vbuf[pl.ds(o, _L)] = vv * scale_v
        ridx[pl.ds(o, _L)] = ebuf[pl.ds(_C + o, _L)]
        for i in range(_L):
          e = o + i
          vb = plsc.load_gather(vbuf, [jnp.full((_L,), e, jnp.int32)])
          rbuf[e, pl.ds(0, _L)] = rbuf[e, pl.ds(0, _L)] * vb
          rbuf[e, pl.ds(_L, _L)] = rbuf[e, pl.ds(_L, _L)] * vb

      pltpu.sync_copy(rbuf, accum.at[ridx], add=True)

  emit_pass(pk_a, tb_base, 0, _N)
  emit_pass(pk_i, tb_img, 1, _N)
  emit_pass(pk_t, tb_txt, 2, _N)
  plsc.subcore_barrier()

  hN = h * _N_PAD

  @pl.loop(0, _N_CH)
  def _modal_out(k):
    base = stripe0 + k * _C
    pltpu.sync_copy(accum.at[pl.ds(base, _C)], wbuf)
    pltpu.sync_copy(wbuf, modal_hbm.at[pl.ds(hN + base, _C)])

    @pl.loop(0, _C)
    def _scale(r):
      wbuf[r, pl.ds(0, _L)] = wbuf[r, pl.ds(0, _L)] * (1.0 + _RESID_W)
      wbuf[r, pl.ds(_L, _L)] = wbuf[r, pl.ds(_L, _L)] * (1.0 + _RESID_W)

    pltpu.sync_copy(wbuf, accum.at[pl.ds(base, _C)])

  plsc.subcore_barrier()

  emit_pass(pk_a, modal_hbm, 3, _N_PAD)
  plsc.subcore_barrier()

  @pl.loop(0, _N_CH)
  def _final_out(k):
    base = stripe0 + k * _C
    pltpu.sync_copy(accum.at[pl.ds(base, _C)], wbuf)
    pltpu.sync_copy(wbuf, out_hbm.at[h, pl.ds(base, _C)])


def _sc_spmm(pk_a, pk_i, pk_t, tb_base, tb_img, tb_txt, scales):
  mesh = plsc.VectorSubcoreMesh(
      core_axis_name="c", subcore_axis_name="s",
      num_cores=_NC, num_subcores=_NS)
  fn = pl.kernel(
      _sc_kernel_body,
      out_type=[
          jax.ShapeDtypeStruct((_NC, _N_PAD, _HD), jnp.float32),
          jax.ShapeDtypeStruct((_NC * _N_PAD, _HD), jnp.float32),
      ],
      mesh=mesh,
      compiler_params=pltpu.CompilerParams(
          needs_layout_passes=False, use_tc_tiling_on_sc=False),
      scratch_types=[
          pltpu.VMEM_SHARED((_N_PAD, _HD), jnp.float32),
          pltpu.VMEM((3 * _C,), jnp.int32),
          pltpu.VMEM((_C,), jnp.int32),
          pltpu.VMEM((_C,), jnp.float32),
          pltpu.VMEM((_C, _HD), jnp.float32),
          pltpu.VMEM((_C, _HD), jnp.float32),
          pltpu.VMEM((4, _L), jnp.float32),
          pltpu.SemaphoreType.DMA,
      ],
  )
  out, _ = fn(pk_a, pk_i, pk_t, tb_base, tb_img, tb_txt, scales)
  return out


def _proj_body(x_ref, w_ref, b_ref, o_ref):
  y = jnp.dot(x_ref[...], w_ref[...], preferred_element_type=jnp.float32)
  y = y + b_ref[...]
  norm = jnp.sqrt(jnp.sum(y * y, axis=1, keepdims=True))
  o_ref[...] = y / jnp.maximum(norm, 1e-12)


def _proj_l2(x, w, b):
  m, k = x.shape
  bm = 400
  return pl.pallas_call(
      _proj_body,
      grid=(m // bm,),
      in_specs=[
          pl.BlockSpec((bm, k), lambda i: (i, 0)),
          pl.BlockSpec((k, _D), lambda i: (0, 0)),
          pl.BlockSpec((1, _D), lambda i: (0, 0)),
      ],
      out_specs=pl.BlockSpec((bm, _D), lambda i: (i, 0)),
      out_shape=jax.ShapeDtypeStruct((m, _D), jnp.float32),
  )(x, w, b.reshape(1, _D))


def _pack_edges(indices, values):
  rows = jnp.pad(indices[0], (0, _E_PAD - _E)).reshape(-1, _C)
  cols = jnp.pad(indices[1], (0, _E_PAD - _E)).reshape(-1, _C)
  vals = jax.lax.bitcast_convert_type(
      jnp.pad(values, (0, _E_PAD - _E)), jnp.int32).reshape(-1, _C)
  return jnp.stack([cols, rows, vals], axis=1).reshape(-1)


def _halves_table(top, bot):
  return jnp.concatenate(
      [top[:, :_HD], bot[:, :_HD], top[:, _HD:], bot[:, _HD:]], axis=0)


def kernel(adj_indices, adj_values, image_adj_indices, image_adj_values,
           text_adj_indices, text_adj_values, u_embs, i_embs,
           image_embedding, text_embedding, image_W, image_b,
           text_W, text_b, modal_weight):
  img_feats = _proj_l2(image_embedding, image_W, image_b)
  txt_feats = _proj_l2(text_embedding, text_W, text_b)

  weight = jax.nn.softmax(modal_weight, axis=-1)
  ones = jnp.ones((_L,), jnp.float32)
  scales = jnp.stack([
      ones,
      _MODAL_ADJ_W * weight[0] * ones,
      _MODAL_ADJ_W * weight[1] * ones,
      ones,
  ])

  tb_base = _halves_table(u_embs, i_embs)
  tb_img = _halves_table(u_embs, img_feats)
  tb_txt = _halves_table(u_embs, txt_feats)

  pk_a = _pack_edges(adj_indices, adj_values)
  pk_i = _pack_edges(image_adj_indices, image_adj_values)
  pk_t = _pack_edges(text_adj_indices, text_adj_values)

  out = _sc_spmm(pk_a, pk_i, pk_t, tb_base, tb_img, tb_txt, scales)

  final = jnp.concatenate([out[0, :_N, :], out[1, :_N, :]], axis=1)
  return final[:_USER_NUM], final[_USER_NUM:]
